# baseline (speedup 1.0000x reference)
import functools

import jax
import jax.numpy as jnp
from jax import lax
from jax.experimental import pallas as pl
from jax.experimental.pallas import tpu as pltpu
from jax.experimental.pallas import tpu_sc as plsc

D = 64
SCALE = 8.0
NW = 32
L = 16
VOCAB = 1000000
NVT = VOCAB // 128
VT_MAIN = 7808
NPAIR = VOCAB // 2


def _iotas():
    i = lax.iota(jnp.int32, L)
    return [i + 16 * g for g in range(4)]


def _mesh():
    return plsc.VectorSubcoreMesh(core_axis_name="c", subcore_axis_name="s")


def _retile():
    @functools.partial(
        pl.kernel,
        mesh=_mesh(),
        compiler_params=pltpu.CompilerParams(use_tc_tiling_on_sc=True),
        out_type=jax.ShapeDtypeStruct((NVT + 1, D, 128), jnp.float32),
        scratch_types=[
            pltpu.VMEM((D, 128), jnp.float32),
            pltpu.VMEM((D, 128), jnp.float32),
            pltpu.SemaphoreType.DMA,
            pltpu.SemaphoreType.DMA,
            pltpu.SemaphoreType.DMA,
        ],
    )
    def k(lut_t, tail128, raw, tbuf0, tbuf1, gsem0, gsem1, osem):
        wid = lax.axis_index("s") * 2 + lax.axis_index("c")
        tbufs = (tbuf0, tbuf1)
        gsems = (gsem0, gsem1)

        def fire_in(vt, b):
            for j in range(8):
                pltpu.async_copy(
                    lut_t.at[pl.ds(8 * j, 8), pl.ds(vt * 128, 128)],
                    tbufs[b].at[pl.ds(8 * j, 8)],
                    gsems[b],
                )

        def drain_in(b):
            pltpu.make_async_copy(
                lut_t.at[pl.ds(0, D), pl.ds(0, 128)], tbufs[b], gsems[b]
            ).wait()

        def fire_out(vt, b):
            pltpu.async_copy(tbufs[b], raw.at[vt], osem)

        def drain_out():
            pltpu.make_async_copy(
                lut_t.at[pl.ds(0, D), pl.ds(0, 128)], tbufs[0], osem
            ).wait()

        vt_base = wid * 244
        fire_in(vt_base, 0)

        def pair_body(t, carry):
            vt0 = vt_base + 2 * t

            @pl.when(t > 0)
            def _():
                drain_out()

            fire_in(vt0 + 1, 1)
            drain_in(0)
            fire_out(vt0, 0)

            @pl.when(t < 121)
            def _():
                drain_out()
                fire_in(vt0 + 2, 0)

            drain_in(1)
            fire_out(vt0 + 1, 1)
            return carry

        lax.fori_loop(0, 122, pair_body, 0)
        drain_out()
        drain_out()

        @pl.when(wid < 4)
        def _():
            vt = VT_MAIN + wid
            fire_in(vt, 0)
            drain_in(0)
            fire_out(vt, 0)
            drain_out()

        @pl.when(wid == 4)
        def _():
            pltpu.sync_copy(tail128, tbufs[0])
            fire_out(NVT, 0)
            drain_out()

    return k


def _transpose_scale():
    @functools.partial(
        pl.kernel,
        mesh=_mesh(),
        compiler_params=pltpu.CompilerParams(
            use_tc_tiling_on_sc=False, needs_layout_passes=False),
        out_type=jax.ShapeDtypeStruct((NPAIR, 128), jnp.float32),
        scratch_types=[
            pltpu.VMEM((D, 128), jnp.float32),
            pltpu.VMEM((D, 128), jnp.float32),
            pltpu.VMEM((D, 128), jnp.float32),
            pltpu.VMEM((D, 128), jnp.float32),
            pltpu.SemaphoreType.DMA,
            pltpu.SemaphoreType.DMA,
            pltpu.SemaphoreType.DMA,
        ],
    )
    def k(raw, dense, tbuf0, tbuf1, obuf0, obuf1, gsem0, gsem1, osem):
        wid = lax.axis_index("s") * 2 + lax.axis_index("c")
        tbufs = (tbuf0, tbuf1)
        obufs = (obuf0, obuf1)
        gsems = (gsem0, gsem1)
        rowv = _iotas()

        def fire_in(vt, b):
            pltpu.async_copy(raw.at[vt], tbufs[b], gsems[b])

        def drain_in(b):
            pltpu.make_async_copy(raw.at[0], tbufs[b], gsems[b]).wait()

        def transpose(b, np):
            @plsc.parallel_loop(0, np, unroll=8)
            def body(p):
                for g in range(8):
                    colv = rowv[0] * 0 + (2 * p + (g // 4))
                    vals = plsc.load_gather(tbufs[b], [rowv[g % 4], colv])
                    obufs[b][p, pl.ds(16 * g, L)] = vals * SCALE

        def fire_out(vt, b, np):
            pltpu.async_copy(
                obufs[b].at[pl.ds(0, np)], dense.at[pl.ds(vt * D, np)], osem
            )

        def drain_out(np):
            pltpu.make_async_copy(
                dense.at[pl.ds(0, np)], obufs[0].at[pl.ds(0, np)], osem
            ).wait()

        vt_base = wid * 244
        fire_in(vt_base, 0)

        def pair_body(t, carry):
            vt0 = vt_base + 2 * t

            @pl.when(t > 0)
            def _():
                drain_out(D)

            fire_in(vt0 + 1, 1)
            drain_in(0)
            transpose(0, D)
            fire_out(vt0, 0, D)

            @pl.when(t < 121)
            def _():
                drain_out(D)
                fire_in(vt0 + 2, 0)

            drain_in(1)
            transpose(1, D)
            fire_out(vt0 + 1, 1, D)
            return carry

        lax.fori_loop(0, 122, pair_body, 0)
        drain_out(D)
        drain_out(D)

        @pl.when(wid < 4)
        def _():
            vt = VT_MAIN + wid
            fire_in(vt, 0)
            drain_in(0)
            transpose(0, D)
            fire_out(vt, 0, D)
            drain_out(D)

        @pl.when(wid == 4)
        def _():
            fire_in(NVT, 0)
            drain_in(0)
            transpose(0, 32)
            fire_out(NVT, 0, 32)
            drain_out(32)

    return k


def _gather(b0, b1):
    rows_per_w = b0 // NW

    @functools.partial(
        pl.kernel,
        mesh=_mesh(),
        compiler_params=pltpu.CompilerParams(
            use_tc_tiling_on_sc=False, needs_layout_passes=False),
        out_type=jax.ShapeDtypeStruct((b1, 8, NW, 8, 128), jnp.float32),
        scratch_types=[
            pltpu.VMEM((rows_per_w, b1), jnp.int32),
            pltpu.VMEM((rows_per_w, 128), jnp.float32),
            pltpu.VMEM((rows_per_w, 128), jnp.float32),
            pltpu.VMEM((D, 128), jnp.float32),
            pltpu.VMEM((D, 128), jnp.float32),
            pltpu.VMEM((2, rows_per_w), jnp.int32),
            pltpu.VMEM((2, rows_per_w), jnp.int32),
            pltpu.SemaphoreType.DMA,
            pltpu.SemaphoreType.DMA,
            pltpu.SemaphoreType.DMA,
        ],
    )
    def k(x_hbm, dense, out5, idx_v, rbuf0, rbuf1, obuf0, obuf1,
          pid_b, hoff_b, gsem0, gsem1, osem):
        wid = lax.axis_index("s") * 2 + lax.axis_index("c")
        rbufs = (rbuf0, rbuf1)
        obufs = (obuf0, obuf1)
        gsems = (gsem0, gsem1)
        rowv = _iotas()
        rowv8 = rowv + [r + D for r in rowv]
        pltpu.sync_copy(x_hbm.at[pl.ds(wid * rows_per_w, rows_per_w)], idx_v)

        def prep(s, b):
            colv = rowv[0] * 0 + s
            for g in range(8):
                ids = plsc.load_gather(idx_v, [rowv8[g], colv])
                pid_b[b, pl.ds(16 * g, L)] = ids >> 1
                hoff_b[b, pl.ds(16 * g, L)] = (ids & 1) << 6

        def fire_gather(b):
            pltpu.async_copy(dense.at[pid_b.at[b]], rbufs[b], gsems[b])

        def drain_gather(b):
            pltpu.make_async_copy(
                dense.at[pl.ds(0, rows_per_w)], rbufs[b], gsems[b]
            ).wait()

        def transpose(b):
            for g in range(8):
                hoffv = hoff_b[b, pl.ds(16 * g, L)]

                @plsc.parallel_loop(0, D // 4, unroll=8)
                def body(d4):
                    for dd in range(4):
                        colv = hoffv + (4 * d4 + dd)
                        vals = plsc.load_gather(rbufs[b], [rowv8[g], colv])
                        obufs[b][4 * d4 + dd, pl.ds(16 * g, L)] = vals

        def fire_writes(s, b):
            for dg in range(8):
                pltpu.async_copy(
                    obufs[b].at[pl.ds(8 * dg, 8)], out5.at[s, dg, wid], osem
                )

        def drain_writes():
            pltpu.make_async_copy(dense.at[pl.ds(0, D)], obufs[0], osem).wait()

        prep(0, 0)
        fire_gather(0)

        def pair_body(t, carry):
            s0 = 2 * t

            @pl.when(t > 0)
            def _():
                drain_writes()

            prep(s0 + 1, 1)
            fire_gather(1)
            drain_gather(0)
            transpose(0)
            fire_writes(s0, 0)

            @pl.when(t < b1 // 2 - 1)
            def _():
                drain_writes()
                prep(s0 + 2, 0)
                fire_gather(0)

            drain_gather(1)
            transpose(1)
            fire_writes(s0 + 1, 1)
            return carry

        lax.fori_loop(0, b1 // 2, pair_body, 0)
        drain_writes()
        drain_writes()

    return k


def kernel(x, lut):
    b0, b1 = x.shape
    tail128 = jnp.pad(lut.T[:, NVT * 128:], ((0, 0), (0, 64)))
    raw = _retile()(lut.T, tail128)
    dense = _transpose_scale()(raw)
    out5 = _gather(b0, b1)(x.astype(jnp.int32), dense)
    return out5.transpose(2, 4, 0, 1, 3).reshape(b0, b1, D)


# speedup vs baseline: 1.5589x; 1.5589x over previous
import functools

import jax
import jax.numpy as jnp
from jax import lax
from jax.experimental import pallas as pl
from jax.experimental.pallas import tpu as pltpu
from jax.experimental.pallas import tpu_sc as plsc

D_MODEL = 64
SCALE = 8.0
NW = 32
L = 16
XROWS = 4
SPLITS = ((0, 104), (104, 96))
RC = XROWS * 200


def _build(b0, b1):
    rows_per_w = b0 // NW
    n_chunks = rows_per_w // XROWS
    mesh = plsc.VectorSubcoreMesh(core_axis_name="c", subcore_axis_name="s")

    @functools.partial(
        pl.kernel,
        mesh=mesh,
        compiler_params=pltpu.CompilerParams(use_tc_tiling_on_sc=False),
        out_type=jax.ShapeDtypeStruct((b0, b1, D_MODEL), jnp.float32),
        scratch_types=[
            pltpu.VMEM((rows_per_w, b1), jnp.int32),
            pltpu.VMEM((2, RC, D_MODEL), jnp.float32),
            pltpu.SemaphoreType.DMA,
            pltpu.SemaphoreType.DMA,
            pltpu.SemaphoreType.DMA,
        ],
    )
    def emb_kernel(x_hbm, lut_hbm, out_hbm, idx_v, bufs, gsem0, gsem1, osem):
        wid = lax.axis_index("s") * 2 + lax.axis_index("c")
        row0 = wid * rows_per_w
        gsems = (gsem0, gsem1)
        pltpu.sync_copy(x_hbm.at[pl.ds(row0, rows_per_w)], idx_v)

        def fire_gathers(s, b):
            for r in range(XROWS):
                for off, sz in SPLITS:
                    pltpu.async_copy(
                        lut_hbm.at[idx_v.at[s * XROWS + r, pl.ds(off, sz)]],
                        bufs.at[b, pl.ds(r * 200 + off, sz)],
                        gsems[b],
                    )

        def drain_gathers(b):
            pltpu.make_async_copy(
                lut_hbm.at[pl.ds(0, RC)], bufs.at[b], gsems[b]
            ).wait()

        def fire_writes(s, b):
            for r in range(XROWS):
                pltpu.async_copy(
                    bufs.at[b, pl.ds(r * 200, 200)],
                    out_hbm.at[row0 + s * XROWS + r],
                    osem,
                )

        def drain_writes():
            pltpu.make_async_copy(lut_hbm.at[pl.ds(0, RC)], bufs.at[0], osem).wait()

        def scale(b):
            def body(i, c):
                r = i * 4
                for dr in range(4):
                    for cc in range(D_MODEL // L):
                        sl = pl.ds(cc * L, L)
                        bufs[b, r + dr, sl] = bufs[b, r + dr, sl] * SCALE
                return c

            lax.fori_loop(0, RC // 4, body, 0)

        fire_gathers(0, 0)
        n_pairs = n_chunks // 2

        def pair_body(t, carry):
            s0 = 2 * t

            @pl.when(t > 0)
            def _():
                drain_writes()

            fire_gathers(s0 + 1, 1)
            drain_gathers(0)
            scale(0)
            fire_writes(s0, 0)

            @pl.when(t < n_pairs - 1)
            def _():
                drain_writes()
                fire_gathers(s0 + 2, 0)

            drain_gathers(1)
            scale(1)
            fire_writes(s0 + 1, 1)
            return carry

        lax.fori_loop(0, n_pairs, pair_body, 0)
        drain_writes()
        drain_writes()

    return emb_kernel


def kernel(x, lut):
    b0, b1 = x.shape
    out = _build(b0, b1)(x.astype(jnp.int32), lut)
    return out
